# Initial kernel scaffold; baseline (speedup 1.0000x reference)
import jax
import jax.numpy as jnp
from jax.experimental import pallas as pl


def kernel(q, K0, V0, sal0, K1, V1, sal1, K2, V2, sal2):
    raise NotImplementedError("write your pallas kernel here")


# speedup vs baseline: 2.7280x; 2.7280x over previous
import functools
import math

import jax
import jax.numpy as jnp
from jax import lax
from jax.experimental import pallas as pl
from jax.experimental.pallas import tpu as pltpu
from jax.experimental.pallas import tpu_sc as plsc

_B, _T, _D = 4, 2048, 1024
_SLOTS = (512, 256, 128)
_SALL = sum(_SLOTS)
_K = 4
_NSEL = _K * len(_SLOTS)
_LANES = 16
_N = _B * _T
_TQ = 256
_NW = 32
_QPW = _N // _NW
_WCOLS = 256


def _score_topk_body(q_ref, k_ref, sal_ref, idx_ref, w_ref):
    q = q_ref[...]
    k = k_ref[...]
    s = lax.dot_general(q, k, (((1,), (1,)), ((), ())),
                        preferred_element_type=jnp.float32)
    s = s * (1.0 / math.sqrt(_D)) + sal_ref[...]

    out_w = []
    out_i = []
    off = 0
    for S in _SLOTS:
        seg = s[:, off:off + S]
        col = lax.broadcasted_iota(jnp.int32, (_TQ, S), 1)
        vals, idxs = [], []
        for _ in range(_K):
            m = jnp.max(seg, axis=1, keepdims=True)
            ij = jnp.min(jnp.where(seg == m, col, S), axis=1, keepdims=True)
            vals.append(m)
            idxs.append(ij)
            seg = jnp.where(col == ij, -jnp.inf, seg)
        m0 = vals[0]
        es = [jnp.exp(v - m0) for v in vals]
        z = (es[0] + es[1]) + (es[2] + es[3])
        inv = (1.0 / len(_SLOTS)) / z
        out_w += [e * inv for e in es]
        out_i += [ij + off for ij in idxs]
        off += S

    pad = _LANES - _NSEL
    out_i += [jnp.zeros((_TQ, 1), jnp.int32)] * pad
    jlane = lax.broadcasted_iota(jnp.int32, (_TQ, _WCOLS), 1) // _LANES
    wrep = jnp.zeros((_TQ, _WCOLS), jnp.float32)
    for j, wj in enumerate(out_w):
        wrep = jnp.where(jlane == j, wj, wrep)
    w_ref[...] = wrep
    idx_ref[...] = jnp.concatenate(out_i, axis=1)


def _scores_topk(qf, k_all, sal_all, interpret=False):
    return pl.pallas_call(
        _score_topk_body,
        grid=(_N // _TQ,),
        in_specs=[
            pl.BlockSpec((_TQ, _D), lambda i: (i, 0)),
            pl.BlockSpec((_SALL, _D), lambda i: (0, 0)),
            pl.BlockSpec((1, _SALL), lambda i: (0, 0)),
        ],
        out_specs=[
            pl.BlockSpec((_TQ, _LANES), lambda i: (i, 0)),
            pl.BlockSpec((_TQ, _WCOLS), lambda i: (i, 0)),
        ],
        out_shape=[
            jax.ShapeDtypeStruct((_N, _LANES), jnp.int32),
            jax.ShapeDtypeStruct((_N, _WCOLS), jnp.float32),
        ],
        interpret=interpret,
    )(qf, k_all, sal_all)


def _sc_body(v_hbm, idx_hbm, w_hbm, out_hbm, idx_v, w_v, rows_v, row_o, sem_g, sem_o):
    wid = lax.axis_index("s") * 2 + lax.axis_index("c")
    base = wid * _QPW
    pltpu.sync_copy(idx_hbm.at[pl.ds(base, _QPW)], idx_v)
    pltpu.sync_copy(w_hbm.at[pl.ds(base, _QPW)], w_v)

    def gather(q):
        return pltpu.make_async_copy(
            v_hbm.at[idx_v.at[q, pl.ds(0, _NSEL)]], rows_v, sem_g)

    def body(q, carry):
        gather(q).start()
        gather(q).wait()
        wbs = [w_v[q, pl.ds(j * _LANES, _LANES)] for j in range(_NSEL)]
        for c in range(_D // _LANES):
            s, l = c // 8, (c % 8) * _LANES
            sl = pl.ds(l, _LANES)
            terms = [wbs[j] * rows_v[j, s, sl] for j in range(_NSEL)]
            while len(terms) > 1:
                terms = [terms[i] + terms[i + 1] if i + 1 < len(terms)
                         else terms[i] for i in range(0, len(terms), 2)]
            row_o[0, s, sl] = terms[0]
        pltpu.make_async_copy(row_o, out_hbm.at[pl.ds(base + q, 1)], sem_o).start()
        pltpu.make_async_copy(row_o, out_hbm.at[pl.ds(base + q, 1)], sem_o).wait()
        return carry

    lax.fori_loop(0, _QPW, body, 0)


def _sc_gather_combine(v_all, idx, w):
    mesh = plsc.VectorSubcoreMesh(core_axis_name="c", subcore_axis_name="s")
    run = pl.kernel(
        _sc_body,
        mesh=mesh,
        out_type=jax.ShapeDtypeStruct((_N, 8, 128), jnp.float32),
        scratch_types=[
            pltpu.VMEM((_QPW, _LANES), jnp.int32),
            pltpu.VMEM((_QPW, _WCOLS), jnp.float32),
            pltpu.VMEM((_NSEL, 8, 128), jnp.float32),
            pltpu.VMEM((1, 8, 128), jnp.float32),
            pltpu.SemaphoreType.DMA,
            pltpu.SemaphoreType.DMA,
        ],
    )
    return run(v_all, idx, w)


def kernel(q, K0, V0, sal0, K1, V1, sal1, K2, V2, sal2):
    qf = q.reshape(_N, _D)
    k_all = jnp.concatenate([K0, K1, K2], axis=0)
    v_all = jnp.concatenate([V0, V1, V2], axis=0)
    sal_all = jnp.concatenate([sal0, sal1, sal2]).reshape(1, _SALL)
    idx, w = _scores_topk(qf, k_all, sal_all)
    out = _sc_gather_combine(v_all.reshape(_SALL, 8, 128), idx, w)
    return out.reshape(_B, _T, _D)


# speedup vs baseline: 5.8707x; 2.1520x over previous
import functools
import math

import jax
import jax.numpy as jnp
from jax import lax
from jax.experimental import pallas as pl
from jax.experimental.pallas import tpu as pltpu
from jax.experimental.pallas import tpu_sc as plsc

_B, _T, _D = 4, 2048, 1024
_SLOTS = (512, 256, 128)
_SALL = sum(_SLOTS)
_K = 4
_NSEL = _K * len(_SLOTS)
_LANES = 16
_N = _B * _T
_TQ = 256
_NW = 32
_QPW = _N // _NW
_WCOLS = 256


def _score_topk_body(q_ref, k_ref, sal_ref, idx_ref, w_ref):
    q = q_ref[...]
    k = k_ref[...]
    s = lax.dot_general(q, k, (((1,), (1,)), ((), ())),
                        preferred_element_type=jnp.float32)
    s = s * (1.0 / math.sqrt(_D)) + sal_ref[...]

    out_w = []
    out_i = []
    off = 0
    for S in _SLOTS:
        seg = s[:, off:off + S]
        col = lax.broadcasted_iota(jnp.int32, (_TQ, S), 1)
        vals, idxs = [], []
        for _ in range(_K):
            m = jnp.max(seg, axis=1, keepdims=True)
            ij = jnp.min(jnp.where(seg == m, col, S), axis=1, keepdims=True)
            vals.append(m)
            idxs.append(ij)
            seg = jnp.where(col == ij, -jnp.inf, seg)
        m0 = vals[0]
        es = [jnp.exp(v - m0) for v in vals]
        z = (es[0] + es[1]) + (es[2] + es[3])
        inv = (1.0 / len(_SLOTS)) / z
        out_w += [e * inv for e in es]
        out_i += [ij + off for ij in idxs]
        off += S

    pad = _LANES - _NSEL
    out_i += [jnp.zeros((_TQ, 1), jnp.int32)] * pad
    jlane = lax.broadcasted_iota(jnp.int32, (_TQ, _WCOLS), 1) // _LANES
    wrep = jnp.zeros((_TQ, _WCOLS), jnp.float32)
    for j, wj in enumerate(out_w):
        wrep = jnp.where(jlane == j, wj, wrep)
    w_ref[...] = wrep
    idx_ref[...] = jnp.concatenate(out_i, axis=1)


def _scores_topk(qf, k_all, sal_all, interpret=False):
    return pl.pallas_call(
        _score_topk_body,
        grid=(_N // _TQ,),
        in_specs=[
            pl.BlockSpec((_TQ, _D), lambda i: (i, 0)),
            pl.BlockSpec((_SALL, _D), lambda i: (0, 0)),
            pl.BlockSpec((1, _SALL), lambda i: (0, 0)),
        ],
        out_specs=[
            pl.BlockSpec((_TQ, _LANES), lambda i: (i, 0)),
            pl.BlockSpec((_TQ, _WCOLS), lambda i: (i, 0)),
        ],
        out_shape=[
            jax.ShapeDtypeStruct((_N, _LANES), jnp.int32),
            jax.ShapeDtypeStruct((_N, _WCOLS), jnp.float32),
        ],
        interpret=interpret,
    )(qf, k_all, sal_all)


_RING = 4


def _sc_body(v_hbm, idx_hbm, w_hbm, out_hbm, idx_v, w_v, rows_v, row_o, *sems):
    sem_g = sems[:_RING]
    sem_o = sems[_RING:]
    wid = lax.axis_index("s") * 2 + lax.axis_index("c")
    base = wid * _QPW
    pltpu.sync_copy(
        idx_hbm.at[pl.ds(pl.multiple_of(base * _LANES, 8), _QPW * _LANES)],
        idx_v)
    pltpu.sync_copy(w_hbm.at[pl.ds(base, _QPW)], w_v)

    def gather(q, par):
        go = pl.multiple_of(q * _LANES, 8)
        return pltpu.make_async_copy(
            v_hbm.at[idx_v.at[pl.ds(go, _NSEL)]], rows_v.at[par], sem_g[par])

    def out_copy(q, par):
        return pltpu.make_async_copy(
            row_o.at[pl.ds(par, 1)], out_hbm.at[pl.ds(base + q, 1)], sem_o[par])

    for par in range(_RING):
        gather(par, par).start()

    def body(g, carry):
        for par in range(_RING):
            q = g * _RING + par
            gather(q, par).wait()
            wbs = [w_v[q, pl.ds(j * _LANES, _LANES)] for j in range(_NSEL)]

            @pl.when(g >= 1)
            def _():
                out_copy(q - _RING, par).wait()

            def chunk(s, c2):
                for l in range(8):
                    sl = pl.ds(l * _LANES, _LANES)
                    terms = [wbs[j] * rows_v[par, j, s, sl] for j in range(_NSEL)]
                    while len(terms) > 1:
                        terms = [terms[i] + terms[i + 1] if i + 1 < len(terms)
                                 else terms[i] for i in range(0, len(terms), 2)]
                    row_o[par, s, sl] = terms[0]
                return c2

            lax.fori_loop(0, 8, chunk, 0)
            out_copy(q, par).start()

            @pl.when(g < _QPW // _RING - 1)
            def _():
                gather(q + _RING, par).start()
        return carry

    lax.fori_loop(0, _QPW // _RING, body, 0)
    for par in range(_RING):
        out_copy(_QPW - _RING + par, par).wait()


def _sc_gather_combine(v_all, idx_packed, w):
    mesh = plsc.VectorSubcoreMesh(core_axis_name="c", subcore_axis_name="s")
    run = pl.kernel(
        _sc_body,
        mesh=mesh,
        out_type=jax.ShapeDtypeStruct((_N, 8, 128), jnp.float32),
        scratch_types=[
            pltpu.VMEM((_QPW * _LANES,), jnp.int32),
            pltpu.VMEM((_QPW, _WCOLS), jnp.float32),
            pltpu.VMEM((_RING, _NSEL, 8, 128), jnp.float32),
            pltpu.VMEM((_RING, 8, 128), jnp.float32),
        ] + [pltpu.SemaphoreType.DMA] * (2 * _RING),
    )
    return run(v_all, idx_packed, w)


def kernel(q, K0, V0, sal0, K1, V1, sal1, K2, V2, sal2):
    qf = q.reshape(_N, _D)
    k_all = jnp.concatenate([K0, K1, K2], axis=0)
    v_all = jnp.concatenate([V0, V1, V2], axis=0)
    sal_all = jnp.concatenate([sal0, sal1, sal2]).reshape(1, _SALL)
    idx, w = _scores_topk(qf, k_all, sal_all)
    out = _sc_gather_combine(v_all.reshape(_SALL, 8, 128),
                             idx.reshape(_N * _LANES), w)
    return out.reshape(_B, _T, _D)
